# baseline (speedup 1.0000x reference)
import functools

import jax
import jax.numpy as jnp
from jax import lax
from jax.experimental import pallas as pl
from jax.experimental.pallas import tpu as pltpu
from jax.experimental.pallas import tpu_sc as plsc

NC = 2
NS = 16
NW = NC * NS
CHUNK_CAP = 80


def _pick_chunk(ept, cap=CHUNK_CAP):
    for ch in range(cap, 7, -8):
        if ept % ch == 0:
            return ch
    raise ValueError(f"no valid edge chunk for {ept} edges per tile")


def _rows_per_tile(n):
    rpt = -(-n // NS)
    return -(-rpt // 128) * 128


def _make_agg(n, dh, e, n_out):
    d = 2 * dh
    ept = e // NS
    ch = _pick_chunk(ept)
    nch = ept // ch
    nbuf = 1
    for cand in range(5, 1, -1):
        if nch % (2 * cand) == 0:
            nbuf = cand
            break
    nit2 = nch // (2 * nbuf)
    rpt = _rows_per_tile(n)
    n_pad = NS * rpt
    zrows = 128
    nz = rpt // zrows

    mesh = plsc.VectorSubcoreMesh(core_axis_name="c", subcore_axis_name="s")

    @functools.partial(
        pl.kernel,
        mesh=mesh,
        compiler_params=pltpu.CompilerParams(use_tc_tiling_on_sc=False),
        out_type=jax.ShapeDtypeStruct((n_out, d), jnp.float32),
        scratch_types=[
            pltpu.VMEM((2, nbuf, ch), jnp.int32),
            pltpu.VMEM((2, nbuf, ch), jnp.int32),
            pltpu.VMEM((2 * nbuf, ch, dh), jnp.float32),
            pltpu.VMEM((zrows, dh), jnp.float32),
            pltpu.VMEM((dh,), jnp.float32),
            pltpu.VMEM_SHARED((n_pad, dh), jnp.float32),
            pltpu.SemaphoreType.DMA,
            pltpu.SemaphoreType.DMA,
            pltpu.SemaphoreType.DMA,
            pltpu.SemaphoreType.DMA,
        ],
    )
    def agg(t_hbm, src_hbm, dst_hbm, bias_hbm, out_hbm, src_v, dst_v, rows_v,
            zbuf_v, bias_v, acc_sh, gsem_a, gsem_b, ssem_a, ssem_b):
        cid = lax.axis_index("c")
        sid = lax.axis_index("s")

        pltpu.sync_copy(bias_hbm.at[cid], bias_v)
        src_t = src_hbm.at[sid]
        dst_t = dst_hbm.at[sid]

        per_row = dh // 16

        def bstore(i, carry):
            j = i % per_row
            zbuf_v[i // per_row, pl.ds(j * 16, 16)] = bias_v[pl.ds(j * 16, 16)]
            return carry

        lax.fori_loop(0, zrows * per_row, bstore, 0)
        base = sid * rpt
        for r in range(nz):
            pltpu.sync_copy(zbuf_v, acc_sh.at[pl.ds(base + r * zrows, zrows)])
        plsc.subcore_barrier()

        half = t_hbm.at[cid]

        def _loop(fn):
            def body(k, carry):
                fn(k)
                return carry
            lax.fori_loop(0, nbuf, body, 0)

        def i_fetch(batch, s):
            pltpu.sync_copy(src_t.at[pl.ds(batch * nbuf, nbuf)],
                            src_v.at[s])
            pltpu.sync_copy(dst_t.at[pl.ds(batch * nbuf, nbuf)],
                            dst_v.at[s])

        def g_start(s, sem):
            _loop(lambda k: pltpu.async_copy(
                half.at[src_v.at[s, k]], rows_v.at[s * nbuf + k], sem))

        def g_wait(s, sem):
            _loop(lambda k: pltpu.make_async_copy(
                half.at[src_v.at[s, k]], rows_v.at[s * nbuf + k], sem).wait())

        def s_start(s, sem):
            _loop(lambda k: pltpu.async_copy(
                rows_v.at[s * nbuf + k],
                acc_sh.at[dst_v.at[s, k]], sem, add=True))

        def s_wait(s, sem):
            _loop(lambda k: pltpu.make_async_copy(
                rows_v.at[s * nbuf + k],
                acc_sh.at[dst_v.at[s, k]], sem).wait())

        i_fetch(0, 0)
        g_start(0, gsem_a)

        def body(i, carry):
            a = 2 * i
            b = 2 * i + 1

            g_wait(0, gsem_a)

            @pl.when(i > 0)
            def _():
                s_wait(1, ssem_b)

            i_fetch(b, 1)
            g_start(1, gsem_b)
            s_start(0, ssem_a)
            g_wait(1, gsem_b)
            s_wait(0, ssem_a)

            @pl.when(i + 1 < nit2)
            def _():
                i_fetch(a + 2, 0)
                g_start(0, gsem_a)

            s_start(1, ssem_b)
            return carry

        lax.fori_loop(0, nit2, body, 0)
        s_wait(1, ssem_b)
        plsc.subcore_barrier()

        col = pl.ds(cid * dh, dh)
        for r in range(nz):
            start = base + r * zrows

            @pl.when(start + zrows <= n_out)
            def _():
                pltpu.sync_copy(acc_sh.at[pl.ds(start, zrows)], zbuf_v)
                pltpu.sync_copy(zbuf_v, out_hbm.at[pl.ds(start, zrows), col])

            tail = n_out % zrows
            if tail:
                @pl.when((start < n_out) & (start + zrows > n_out))
                def _():
                    pltpu.sync_copy(acc_sh.at[pl.ds(start, tail)],
                                    zbuf_v.at[pl.ds(0, tail)])
                    pltpu.sync_copy(zbuf_v.at[pl.ds(0, tail)],
                                    out_hbm.at[pl.ds(start, tail), col])

    return agg


def _mm_body(x_ref, w_ref, o_ref):
    r = jnp.dot(x_ref[...], w_ref[...], preferred_element_type=jnp.float32)
    dh = o_ref.shape[2]
    o_ref[0] = r[:, :dh]
    o_ref[1] = r[:, dh:]


def _matmul_split(x, w, blk):
    n, d = x.shape
    dout = w.shape[1]
    dh = dout // 2
    return pl.pallas_call(
        _mm_body,
        grid=(n // blk,),
        in_specs=[
            pl.BlockSpec((blk, d), lambda i: (i, 0)),
            pl.BlockSpec((d, dout), lambda i: (0, 0)),
        ],
        out_specs=pl.BlockSpec((2, blk, dh), lambda i: (0, i, 0)),
        out_shape=jax.ShapeDtypeStruct((2, n, dh), jnp.float32),
    )(x, w)


def _fused_body(p_ref, w_ref, o_ref):
    h = jnp.maximum(p_ref[...], 0.0)
    r = jnp.dot(h, w_ref[...], preferred_element_type=jnp.float32)
    dh = o_ref.shape[2]
    o_ref[0] = r[:, :dh]
    o_ref[1] = r[:, dh:]


def _fused_relu_mm_split(p, w, n, blk):
    _, d = p.shape
    dout = w.shape[1]
    return pl.pallas_call(
        _fused_body,
        grid=(n // blk,),
        in_specs=[
            pl.BlockSpec((blk, d), lambda i: (i, 0)),
            pl.BlockSpec((d, dout), lambda i: (0, 0)),
        ],
        out_specs=pl.BlockSpec((2, blk, dout // 2), lambda i: (0, i, 0)),
        out_shape=jax.ShapeDtypeStruct((2, n, dout // 2), jnp.float32),
    )(p, w)


def kernel(features, edge_index, W1, b1, W2, b2):
    n, d = features.shape
    e = edge_index.shape[1]
    ept = e // NS
    ch = _pick_chunk(ept)
    nch = ept // ch

    src = edge_index[0].reshape(NS, nch, ch)
    dst = edge_index[1].reshape(NS, nch, ch)
    dh = d // 2
    n_pad = NS * _rows_per_tile(n)
    agg1 = _make_agg(n, dh, e, n_pad)
    agg2 = _make_agg(n, dh, e, n)

    blk = n if n % 8 == 0 and n * d * 4 <= 8_000_000 else 2000
    b1s = b1.reshape(NC, dh)
    b2s = b2.reshape(NC, dh)
    t1 = _matmul_split(features, W1, blk)
    p = agg1(t1, src, dst, b1s)
    t2 = _fused_relu_mm_split(p, W2, n, blk)
    return agg2(t2, src, dst, b2s)


# speedup vs baseline: 1.5133x; 1.5133x over previous
import functools

import jax
import jax.numpy as jnp
from jax import lax
from jax.experimental import pallas as pl
from jax.experimental.pallas import tpu as pltpu
from jax.experimental.pallas import tpu_sc as plsc

NC = 2
NS = 16
NW = NC * NS
CHUNK_CAP = 40


def _pick_chunk(ept, cap=CHUNK_CAP):
    for ch in range(cap, 7, -8):
        if ept % ch == 0:
            return ch
    raise ValueError(f"no valid edge chunk for {ept} edges per tile")


def _rows_per_tile(n):
    rpt = -(-n // NS)
    return -(-rpt // 128) * 128


def _make_agg(n, dh, e, n_out):
    d = 2 * dh
    ept = e // NS
    ch = _pick_chunk(ept)
    nch = ept // ch
    nbuf = 1
    for cand in range(5, 1, -1):
        if nch % (2 * cand) == 0:
            nbuf = cand
            break
    nit2 = nch // (2 * nbuf)
    rpt = _rows_per_tile(n)
    n_pad = NS * rpt
    zrows = 128
    nz = rpt // zrows

    mesh = plsc.VectorSubcoreMesh(core_axis_name="c", subcore_axis_name="s")

    @functools.partial(
        pl.kernel,
        mesh=mesh,
        compiler_params=pltpu.CompilerParams(use_tc_tiling_on_sc=False),
        out_type=jax.ShapeDtypeStruct((n_out, d), jnp.float32),
        scratch_types=[
            pltpu.VMEM((nch, ch), jnp.int32),
            pltpu.VMEM((nch, ch), jnp.int32),
            pltpu.VMEM((2 * nbuf, ch, dh), jnp.float32),
            pltpu.VMEM((zrows, dh), jnp.float32),
            pltpu.VMEM((dh,), jnp.float32),
            pltpu.VMEM_SHARED((n_pad, dh), jnp.float32),
            pltpu.SemaphoreType.DMA,
            pltpu.SemaphoreType.DMA,
            pltpu.SemaphoreType.DMA,
            pltpu.SemaphoreType.DMA,
        ],
    )
    def agg(t_hbm, src_hbm, dst_hbm, bias_hbm, out_hbm, src_v, dst_v, rows_v,
            zbuf_v, bias_v, acc_sh, gsem_a, gsem_b, ssem_a, ssem_b):
        cid = lax.axis_index("c")
        sid = lax.axis_index("s")

        pltpu.sync_copy(src_hbm.at[sid], src_v)
        pltpu.sync_copy(dst_hbm.at[sid], dst_v)
        pltpu.sync_copy(bias_hbm.at[cid], bias_v)

        per_row = dh // 16

        def bstore(i, carry):
            j = i % per_row
            zbuf_v[i // per_row, pl.ds(j * 16, 16)] = bias_v[pl.ds(j * 16, 16)]
            return carry

        lax.fori_loop(0, zrows * per_row, bstore, 0)
        base = sid * rpt
        for r in range(nz):
            pltpu.sync_copy(zbuf_v, acc_sh.at[pl.ds(base + r * zrows, zrows)])
        plsc.subcore_barrier()

        half = t_hbm.at[cid]

        def _loop(fn):
            def body(k, carry):
                fn(k)
                return carry
            lax.fori_loop(0, nbuf, body, 0)

        def g_start(batch, s, sem):
            _loop(lambda k: pltpu.async_copy(
                half.at[src_v.at[batch * nbuf + k]],
                rows_v.at[s * nbuf + k], sem))

        def gs_pipe(batch, s, gsem, ssem):
            def one(k):
                pltpu.make_async_copy(
                    half.at[src_v.at[batch * nbuf + k]],
                    rows_v.at[s * nbuf + k], gsem).wait()
                pltpu.async_copy(rows_v.at[s * nbuf + k],
                                 acc_sh.at[dst_v.at[batch * nbuf + k]],
                                 ssem, add=True)
            _loop(one)

        def s_wait(batch, s, sem):
            _loop(lambda k: pltpu.make_async_copy(
                rows_v.at[s * nbuf + k],
                acc_sh.at[dst_v.at[batch * nbuf + k]], sem).wait())

        g_start(0, 0, gsem_a)

        def body(i, carry):
            a = 2 * i
            b = 2 * i + 1

            @pl.when(i > 0)
            def _():
                s_wait(a - 1, 1, ssem_b)

            g_start(b, 1, gsem_b)
            gs_pipe(a, 0, gsem_a, ssem_a)
            s_wait(a, 0, ssem_a)

            @pl.when(i + 1 < nit2)
            def _():
                g_start(a + 2, 0, gsem_a)

            gs_pipe(b, 1, gsem_b, ssem_b)
            return carry

        lax.fori_loop(0, nit2, body, 0)
        s_wait(2 * nit2 - 1, 1, ssem_b)
        plsc.subcore_barrier()

        col = pl.ds(cid * dh, dh)
        for r in range(nz):
            start = base + r * zrows

            @pl.when(start + zrows <= n_out)
            def _():
                pltpu.sync_copy(acc_sh.at[pl.ds(start, zrows)], zbuf_v)
                pltpu.sync_copy(zbuf_v, out_hbm.at[pl.ds(start, zrows), col])

            tail = n_out % zrows
            if tail:
                @pl.when((start < n_out) & (start + zrows > n_out))
                def _():
                    pltpu.sync_copy(acc_sh.at[pl.ds(start, tail)],
                                    zbuf_v.at[pl.ds(0, tail)])
                    pltpu.sync_copy(zbuf_v.at[pl.ds(0, tail)],
                                    out_hbm.at[pl.ds(start, tail), col])

    return agg


def _mm_body(x_ref, w_ref, o_ref):
    r = jnp.dot(x_ref[...], w_ref[...], preferred_element_type=jnp.float32)
    dh = o_ref.shape[2]
    o_ref[0] = r[:, :dh]
    o_ref[1] = r[:, dh:]


def _matmul_split(x, w, blk):
    n, d = x.shape
    dout = w.shape[1]
    dh = dout // 2
    return pl.pallas_call(
        _mm_body,
        grid=(n // blk,),
        in_specs=[
            pl.BlockSpec((blk, d), lambda i: (i, 0)),
            pl.BlockSpec((d, dout), lambda i: (0, 0)),
        ],
        out_specs=pl.BlockSpec((2, blk, dh), lambda i: (0, i, 0)),
        out_shape=jax.ShapeDtypeStruct((2, n, dh), jnp.float32),
    )(x, w)


def _fused_body(p_ref, w_ref, o_ref):
    h = jnp.maximum(p_ref[...], 0.0)
    r = jnp.dot(h, w_ref[...], preferred_element_type=jnp.float32)
    dh = o_ref.shape[2]
    o_ref[0] = r[:, :dh]
    o_ref[1] = r[:, dh:]


def _fused_relu_mm_split(p, w, n, blk):
    _, d = p.shape
    dout = w.shape[1]
    return pl.pallas_call(
        _fused_body,
        grid=(n // blk,),
        in_specs=[
            pl.BlockSpec((blk, d), lambda i: (i, 0)),
            pl.BlockSpec((d, dout), lambda i: (0, 0)),
        ],
        out_specs=pl.BlockSpec((2, blk, dout // 2), lambda i: (0, i, 0)),
        out_shape=jax.ShapeDtypeStruct((2, n, dout // 2), jnp.float32),
    )(p, w)


def kernel(features, edge_index, W1, b1, W2, b2):
    n, d = features.shape
    e = edge_index.shape[1]
    ept = e // NS
    ch = _pick_chunk(ept)
    nch = ept // ch

    src = edge_index[0].reshape(NS, nch, ch)
    dst = edge_index[1].reshape(NS, nch, ch)
    dh = d // 2
    n_pad = NS * _rows_per_tile(n)
    agg1 = _make_agg(n, dh, e, n_pad)
    agg2 = _make_agg(n, dh, e, n)

    blk = n if n % 8 == 0 and n * d * 4 <= 8_000_000 else 2000
    b1s = b1.reshape(NC, dh)
    b2s = b2.reshape(NC, dh)
    t1 = _matmul_split(features, W1, blk)
    p = agg1(t1, src, dst, b1s)
    t2 = _fused_relu_mm_split(p, W2, n, blk)
    return agg2(t2, src, dst, b2s)
